# baseline (speedup 1.0000x reference)
import functools

import jax
import jax.numpy as jnp
from jax import lax
from jax.experimental import pallas as pl
from jax.experimental.pallas import tpu as pltpu
from jax.experimental.pallas import tpu_sc as plsc


def _cumsum_lanes(m, width):
    x = m
    shift = 1
    while shift < width:
        rows = x.shape[0]
        pad = jnp.zeros((rows, shift), jnp.int32)
        x = x + jnp.concatenate([pad, x[:, : width - shift]], axis=1)
        shift *= 2
    return x


def _select_body(s_ref, idx_ref, srcid_ref, *, B, L, K, jblk, npad):
    sc = s_ref[...] + 0.0
    u = lax.bitcast_convert_type(sc, jnp.int32)
    key = jnp.where(u >= 0, u, u ^ jnp.int32(0x7FFFFFFF))
    ukey = lax.bitcast_convert_type(key ^ jnp.int32(-2147483648),
                                    jnp.uint32)

    thr = jnp.zeros((B, 1), jnp.uint32)
    for bit in range(31, -1, -1):
        cand = thr | jnp.uint32(1 << bit)
        cnt = jnp.sum((ukey >= cand).astype(jnp.int32), axis=1,
                      keepdims=True)
        thr = jnp.where(cnt >= K, cand, thr)

    gt = (ukey > thr).astype(jnp.int32)
    eq = (ukey == thr).astype(jnp.int32)
    r = K - jnp.sum(gt, axis=1, keepdims=True)
    eqrank = _cumsum_lanes(eq, L)
    sel = gt | (eq & (eqrank <= r).astype(jnp.int32))
    csel = _cumsum_lanes(sel, L)

    pos = lax.broadcasted_iota(jnp.int32, (B, L), 1)
    boff = lax.broadcasted_iota(jnp.int32, (B, L), 0) * K
    zid = B * K + (pos & (npad - 1))
    srcid_ref[...] = jnp.where(sel == 1, boff + csel - 1, zid)

    for bb in range(B):
        csel_col = csel[bb : bb + 1, :].T
        for jb in range(K // jblk):
            jv = (lax.broadcasted_iota(jnp.int32, (1, jblk), 1)
                  + jb * jblk)
            m = (csel_col <= jv).astype(jnp.int32)
            vals = jnp.sum(m, axis=0) + bb * L
            idx_ref[bb, pl.ds(jb * jblk, jblk)] = vals


def _select(scores, B, L, K, npad):
    jblk = 512
    body = functools.partial(_select_body, B=B, L=L, K=K, jblk=jblk,
                             npad=npad)
    return pl.pallas_call(
        body,
        out_shape=[
            jax.ShapeDtypeStruct((B, K), jnp.int32),
            jax.ShapeDtypeStruct((B, L), jnp.int32),
        ],
    )(scores)


def _gather(x2d, idxf, D, npad=0, zrows=None):
    n = idxf.shape[0]
    mesh = plsc.VectorSubcoreMesh(core_axis_name="c", subcore_axis_name="s")
    info = plsc.get_sparse_core_info()
    nw = info.num_cores * info.num_subcores
    rpw = n // nw
    ch = 64
    nch = rpw // ch
    zpw = npad // nw

    @functools.partial(
        pl.kernel,
        mesh=mesh,
        out_type=jax.ShapeDtypeStruct((n + npad, D), jnp.float32),
        scratch_types=[
            pltpu.VMEM((ch,), jnp.int32),
            pltpu.VMEM((ch, D), jnp.float32),
            pltpu.SemaphoreType.DMA,
        ],
    )
    def gk(x_hbm, idx_hbm, z_hbm, out_hbm, idx_v, rows_v, sem):
        wid = lax.axis_index("s") * info.num_cores + lax.axis_index("c")
        base = wid * rpw
        for c in range(nch):
            off = base + c * ch
            pltpu.sync_copy(idx_hbm.at[pl.ds(off, ch)], idx_v)
            pltpu.async_copy(x_hbm.at[idx_v], rows_v, sem).wait()
            pltpu.sync_copy(rows_v, out_hbm.at[pl.ds(off, ch)])
        if npad:
            pltpu.sync_copy(z_hbm, rows_v.at[pl.ds(0, zpw)])
            pltpu.sync_copy(rows_v.at[pl.ds(0, zpw)],
                            out_hbm.at[pl.ds(n + wid * zpw, zpw)])

    if zrows is None:
        zrows = jnp.zeros((max(zpw, 1), D), jnp.float32)
    return gk(x2d, idxf, zrows)


def _mlp_body(x_ref, w1_ref, b1_ref, w2_ref, b2_ref, o_ref):
    xb = x_ref[...].astype(jnp.bfloat16)
    dff = w1_ref.shape[1]
    cdff = dff // 4
    o = jnp.zeros(o_ref.shape, jnp.float32) + b2_ref[...]
    for c in range(dff // cdff):
        h = jnp.dot(xb, w1_ref[:, pl.ds(c * cdff, cdff)],
                    preferred_element_type=jnp.float32)
        h = h + b1_ref[:, pl.ds(c * cdff, cdff)]
        g = jax.nn.gelu(h).astype(jnp.bfloat16)
        o = o + jnp.dot(g, w2_ref[pl.ds(c * cdff, cdff), :],
                        preferred_element_type=jnp.float32)
    o_ref[...] = o


def _mlp(xsel, w1, b1_row, w2, b2_row, nrows):
    n, D = xsel.shape
    DFF = w1.shape[1]
    rblk = 512
    grid = (nrows // rblk,)
    return pl.pallas_call(
        _mlp_body,
        grid=grid,
        in_specs=[
            pl.BlockSpec((rblk, D), lambda i: (i, 0)),
            pl.BlockSpec((D, DFF), lambda i: (0, 0)),
            pl.BlockSpec((1, DFF), lambda i: (0, 0)),
            pl.BlockSpec((DFF, D), lambda i: (0, 0)),
            pl.BlockSpec((1, D), lambda i: (0, 0)),
        ],
        out_specs=pl.BlockSpec((rblk, D), lambda i: (i, 0)),
        out_shape=jax.ShapeDtypeStruct((n, D), jnp.float32),
        input_output_aliases={0: 0},
    )(xsel, w1, b1_row, w2, b2_row)


def kernel(x, Wr, br, W1, b1, W2, b2):
    B, L, D = x.shape
    DFF = W1.shape[1]
    K = L // 2
    NPAD = 64

    x2d = x.reshape(B * L, D)
    scores = jnp.einsum('bld,d->bl', x, Wr) + br
    idxg, srcid = _select(scores, B, L, K, NPAD)
    idxf = idxg.reshape(B * K)
    srcidf = srcid.reshape(B * L)

    xsel = _gather(x2d, idxf, D, npad=NPAD)
    combo = _mlp(xsel, W1.astype(jnp.bfloat16), b1.reshape(1, DFF),
                 W2.astype(jnp.bfloat16), b2.reshape(1, D), B * K)
    out2d = _gather(combo, srcidf, D)
    return out2d.reshape(B, L, D)


# speedup vs baseline: 1.0341x; 1.0341x over previous
import functools

import jax
import jax.numpy as jnp
from jax import lax
from jax.experimental import pallas as pl
from jax.experimental.pallas import tpu as pltpu
from jax.experimental.pallas import tpu_sc as plsc


def _cumsum_lanes(m, width):
    x = m
    shift = 1
    while shift < width:
        rows = x.shape[0]
        pad = jnp.zeros((rows, shift), jnp.int32)
        x = x + jnp.concatenate([pad, x[:, : width - shift]], axis=1)
        shift *= 2
    return x


def _select_body(s_ref, idx_ref, srcid_ref, *, B, L, K, jblk, npad):
    sc = s_ref[...] + 0.0
    u = lax.bitcast_convert_type(sc, jnp.int32)
    key = jnp.where(u >= 0, u, u ^ jnp.int32(0x7FFFFFFF))
    ukey = lax.bitcast_convert_type(key ^ jnp.int32(-2147483648),
                                    jnp.uint32)

    thr = jnp.zeros((B, 1), jnp.uint32)
    for bit in range(31, -1, -1):
        cand = thr | jnp.uint32(1 << bit)
        cnt = jnp.sum((ukey >= cand).astype(jnp.int32), axis=1,
                      keepdims=True)
        thr = jnp.where(cnt >= K, cand, thr)

    gt = (ukey > thr).astype(jnp.int32)
    eq = (ukey == thr).astype(jnp.int32)
    r = K - jnp.sum(gt, axis=1, keepdims=True)
    eqrank = _cumsum_lanes(eq, L)
    sel = gt | (eq & (eqrank <= r).astype(jnp.int32))
    csel = _cumsum_lanes(sel, L)

    pos = lax.broadcasted_iota(jnp.int32, (B, L), 1)
    boff = lax.broadcasted_iota(jnp.int32, (B, L), 0) * K
    zid = B * K + (pos & (npad - 1))
    srcid_ref[...] = jnp.where(sel == 1, boff + csel - 1, zid)

    for bb in range(B):
        csel_col = csel[bb : bb + 1, :].T
        for jb in range(K // jblk):
            jv = (lax.broadcasted_iota(jnp.int32, (1, jblk), 1)
                  + jb * jblk)
            m = (csel_col <= jv).astype(jnp.int32)
            vals = jnp.sum(m, axis=0) + bb * L
            idx_ref[bb, pl.ds(jb * jblk, jblk)] = vals


def _select(scores, B, L, K, npad):
    jblk = 512
    body = functools.partial(_select_body, B=B, L=L, K=K, jblk=jblk,
                             npad=npad)
    return pl.pallas_call(
        body,
        out_shape=[
            jax.ShapeDtypeStruct((B, K), jnp.int32),
            jax.ShapeDtypeStruct((B, L), jnp.int32),
        ],
    )(scores)


def _gather(x2d, idxf, D, npad=0, zrows=None):
    n = idxf.shape[0]
    mesh = plsc.VectorSubcoreMesh(core_axis_name="c", subcore_axis_name="s")
    info = plsc.get_sparse_core_info()
    nw = info.num_cores * info.num_subcores
    rpw = n // nw
    ch = 64
    nch = rpw // ch
    zpw = npad // nw

    @functools.partial(
        pl.kernel,
        mesh=mesh,
        out_type=jax.ShapeDtypeStruct((n + npad, D), jnp.float32),
        scratch_types=[
            pltpu.VMEM((ch,), jnp.int32),
            pltpu.VMEM((ch, D), jnp.float32),
            pltpu.SemaphoreType.DMA,
        ],
    )
    def gk(x_hbm, idx_hbm, z_hbm, out_hbm, idx_v, rows_v, sem):
        wid = lax.axis_index("s") * info.num_cores + lax.axis_index("c")
        base = wid * rpw
        for c in range(nch):
            off = base + c * ch
            pltpu.sync_copy(idx_hbm.at[pl.ds(off, ch)], idx_v)
            pltpu.async_copy(x_hbm.at[idx_v], rows_v, sem).wait()
            pltpu.sync_copy(rows_v, out_hbm.at[pl.ds(off, ch)])
        if npad:
            pltpu.sync_copy(z_hbm, rows_v.at[pl.ds(0, zpw)])
            pltpu.sync_copy(rows_v.at[pl.ds(0, zpw)],
                            out_hbm.at[pl.ds(n + wid * zpw, zpw)])

    if zrows is None:
        zrows = jnp.zeros((max(zpw, 1), D), jnp.float32)
    return gk(x2d, idxf, zrows)


def _mlp_body(x_ref, w1_ref, b1_ref, w2_ref, b2_ref, o_ref):
    xb = x_ref[...].astype(jnp.bfloat16)
    dff = w1_ref.shape[1]
    cdff = dff // 4
    o = jnp.zeros(o_ref.shape, jnp.float32) + b2_ref[...]
    for c in range(dff // cdff):
        h = jnp.dot(xb, w1_ref[:, pl.ds(c * cdff, cdff)],
                    preferred_element_type=jnp.float32)
        h = h + b1_ref[:, pl.ds(c * cdff, cdff)]
        g = jax.nn.gelu(h).astype(jnp.bfloat16)
        o = o + jnp.dot(g, w2_ref[pl.ds(c * cdff, cdff), :],
                        preferred_element_type=jnp.float32)
    o_ref[...] = o


def _mlp(xsel, w1, b1_row, w2, b2_row, nrows):
    n, D = xsel.shape
    DFF = w1.shape[1]
    rblk = 1024
    grid = (nrows // rblk,)
    return pl.pallas_call(
        _mlp_body,
        grid=grid,
        in_specs=[
            pl.BlockSpec((rblk, D), lambda i: (i, 0)),
            pl.BlockSpec((D, DFF), lambda i: (0, 0)),
            pl.BlockSpec((1, DFF), lambda i: (0, 0)),
            pl.BlockSpec((DFF, D), lambda i: (0, 0)),
            pl.BlockSpec((1, D), lambda i: (0, 0)),
        ],
        out_specs=pl.BlockSpec((rblk, D), lambda i: (i, 0)),
        out_shape=jax.ShapeDtypeStruct((n, D), jnp.float32),
        input_output_aliases={0: 0},
    )(xsel, w1, b1_row, w2, b2_row)


def kernel(x, Wr, br, W1, b1, W2, b2):
    B, L, D = x.shape
    DFF = W1.shape[1]
    K = L // 2
    NPAD = 512

    x2d = x.reshape(B * L, D)
    scores = jnp.einsum('bld,d->bl', x, Wr) + br
    idxg, srcid = _select(scores, B, L, K, NPAD)
    idxf = idxg.reshape(B * K)
    srcidf = srcid.reshape(B * L)

    xsel = _gather(x2d, idxf, D, npad=NPAD)
    combo = _mlp(xsel, W1.astype(jnp.bfloat16), b1.reshape(1, DFF),
                 W2.astype(jnp.bfloat16), b2.reshape(1, D), B * K)
    out2d = _gather(combo, srcidf, D)
    return out2d.reshape(B, L, D)
